# baseline (speedup 1.0000x reference)
import jax
import jax.numpy as jnp
from jax import lax
from jax.experimental import pallas as pl
from jax.experimental.pallas import tpu as pltpu
from jax.experimental.pallas import tpu_sc as plsc

_D = 1024
_B = 8192
_NC = 2
_NS = 16
_NW = _NC * _NS
_BPW = _B // _NW
_CH = 8
_NCHUNK = _BPW // _CH
_NBUF = 14
_RA = 12


def _sc_gather(idx_hbm, table_hbm, out_hbm, idx_v, rows_v, *sems):
    s_in = sems[:_NBUF]
    s_out = sems[_NBUF:]
    wid = lax.axis_index("s") * _NC + lax.axis_index("c")
    base = wid * _BPW
    pltpu.sync_copy(idx_hbm.at[pl.ds(base, _BPW)], idx_v)

    gathers = [None] * _NBUF
    stores = [None] * _NBUF

    def gather(c):
        b = c % _NBUF
        gathers[b] = pltpu.async_copy(
            table_hbm.at[idx_v.at[pl.ds(c * _CH, _CH)]], rows_v.at[b],
            s_in[b])

    for c in range(min(_RA, _NCHUNK)):
        gather(c)
    for c in range(_NCHUNK):
        b = c % _NBUF
        gathers[b].wait()
        stores[b] = pltpu.async_copy(
            rows_v.at[b], out_hbm.at[pl.ds(base + c * _CH, _CH)], s_out[b])
        n = c + _RA
        if n < _NCHUNK:
            bn = n % _NBUF
            if stores[bn] is not None:
                stores[bn].wait()
            gather(n)
    for c in range(max(0, _NCHUNK - _NBUF), _NCHUNK):
        stores[c % _NBUF].wait()


@jax.jit
def _gather(idx, table):
    mesh = plsc.VectorSubcoreMesh(core_axis_name="c", subcore_axis_name="s")
    return pl.kernel(
        _sc_gather,
        mesh=mesh,
        out_type=jax.ShapeDtypeStruct((_B, _D), jnp.float32),
        scratch_types=[
            pltpu.VMEM((_BPW,), jnp.int32),
            pltpu.VMEM((_NBUF, _CH, _D), jnp.float32),
        ] + [pltpu.SemaphoreType.DMA] * (2 * _NBUF),
    )(idx, table)


def kernel(positions, P):
    idx = positions[0].astype(jnp.int32)
    out = _gather(idx, P[0])
    return out[None]


# speedup vs baseline: 1.0080x; 1.0080x over previous
import jax
import jax.numpy as jnp
from jax import lax
from jax.experimental import pallas as pl
from jax.experimental.pallas import tpu as pltpu
from jax.experimental.pallas import tpu_sc as plsc

_D = 1024
_B = 8192
_NC = 2
_NS = 16
_NW = _NC * _NS
_BPW = _B // _NW
_CH = 16
_NCHUNK = _BPW // _CH
_NBUF = 7
_RA = 6


def _sc_gather(idx_hbm, table_hbm, out_hbm, idx_v, rows_v, *sems):
    s_in = sems[:_NBUF]
    s_out = sems[_NBUF:]
    wid = lax.axis_index("s") * _NC + lax.axis_index("c")
    base = wid * _BPW
    pltpu.sync_copy(idx_hbm.at[pl.ds(base, _BPW)], idx_v)

    gathers = [None] * _NBUF
    stores = [None] * _NBUF

    def gather(c):
        b = c % _NBUF
        gathers[b] = pltpu.async_copy(
            table_hbm.at[idx_v.at[pl.ds(c * _CH, _CH)]], rows_v.at[b],
            s_in[b])

    for c in range(min(_RA, _NCHUNK)):
        gather(c)
    for c in range(_NCHUNK):
        b = c % _NBUF
        gathers[b].wait()
        stores[b] = pltpu.async_copy(
            rows_v.at[b], out_hbm.at[pl.ds(base + c * _CH, _CH)], s_out[b])
        n = c + _RA
        if n < _NCHUNK:
            bn = n % _NBUF
            if stores[bn] is not None:
                stores[bn].wait()
            gather(n)
    for c in range(max(0, _NCHUNK - _NBUF), _NCHUNK):
        stores[c % _NBUF].wait()


@jax.jit
def _gather(idx, table):
    mesh = plsc.VectorSubcoreMesh(core_axis_name="c", subcore_axis_name="s")
    return pl.kernel(
        _sc_gather,
        mesh=mesh,
        out_type=jax.ShapeDtypeStruct((_B, _D), jnp.float32),
        scratch_types=[
            pltpu.VMEM((_BPW,), jnp.int32),
            pltpu.VMEM((_NBUF, _CH, _D), jnp.float32),
        ] + [pltpu.SemaphoreType.DMA] * (2 * _NBUF),
    )(idx, table)


def kernel(positions, P):
    idx = positions[0].astype(jnp.int32)
    out = _gather(idx, P[0])
    return out[None]
